# Initial kernel scaffold; baseline (speedup 1.0000x reference)
import jax
import jax.numpy as jnp
from jax.experimental import pallas as pl


def kernel(loc_data, conf_data, prior_data):
    raise NotImplementedError("write your pallas kernel here")


# speedup vs baseline: 172.4970x; 172.4970x over previous
import functools

import jax
import jax.numpy as jnp
from jax import lax
from jax.experimental import pallas as pl
from jax.experimental.pallas import tpu as pltpu

NUM_CLASSES = 2
TOP_K = 750
NMS_THRESH = 0.3
CONF_THRESH = 0.05
NMS_TOP_K = 5000
NUM_PRIORS = 20000

_ROWS = 160
_PAD_N = _ROWS * 128


def _nms_body(loc_ref, sc_ref, pr_ref, out_ref, alive_ref):
    f32 = jnp.float32
    l0 = loc_ref[0, 0]
    l1 = loc_ref[0, 1]
    l2 = loc_ref[0, 2]
    l3 = loc_ref[0, 3]
    pcx = pr_ref[0]
    pcy = pr_ref[1]
    pw = pr_ref[2]
    ph = pr_ref[3]
    cx = pcx + (l0 * f32(0.1)) * pw
    cy = pcy + (l1 * f32(0.1)) * ph
    w = pw * jnp.exp(l2 * f32(0.2))
    h = ph * jnp.exp(l3 * f32(0.2))
    x1 = cx + (-(w / f32(2.0)))
    y1 = cy + (-(h / f32(2.0)))
    x2 = w + x1
    y2 = h + y1
    area = (x2 - x1) * (y2 - y1)

    scores = sc_ref[0, 0]
    idx = (lax.broadcasted_iota(jnp.int32, (_ROWS, 128), 0) * 128
           + lax.broadcasted_iota(jnp.int32, (_ROWS, 128), 1))
    in_range = idx < NUM_PRIORS
    valid = (scores > f32(CONF_THRESH)) & in_range

    sbits = lax.bitcast_convert_type(scores, jnp.int32)
    sb = jnp.where(valid, sbits, jnp.int32(-1))

    def _bs_bits(_, lohi):
        lo, hi = lohi
        mid = (lo + hi) // 2
        cnt = jnp.sum(jnp.where(sb > mid, jnp.int32(1), jnp.int32(0)))
        pred = cnt < NMS_TOP_K
        return (jnp.where(pred, lo, mid + 1), jnp.where(pred, mid, hi))

    _, t_bits = lax.fori_loop(0, 31, _bs_bits, (jnp.int32(0), jnp.int32(0x7F800000)))
    c_gt = jnp.sum(jnp.where(sb > t_bits, jnp.int32(1), jnp.int32(0)))
    r = NMS_TOP_K - c_gt
    eq = sb == t_bits

    def _bs_idx(_, lohi):
        lo, hi = lohi
        mid = (lo + hi) // 2
        cnt = jnp.sum(jnp.where(eq & (idx >= mid), jnp.int32(1), jnp.int32(0)))
        pred = cnt <= r
        return (jnp.where(pred, lo, mid + 1), jnp.where(pred, mid, hi))

    _, bound = lax.fori_loop(0, 16, _bs_idx, (jnp.int32(0), jnp.int32(_PAD_N)))
    participate = valid & ((sb > t_bits) | (eq & (idx >= bound)))

    alive0 = jnp.where(participate, f32(1.0), f32(0.0))
    alive_ref[...] = alive0
    out_ref[...] = jnp.zeros_like(out_ref)

    go0 = jnp.max(alive0) > f32(0.0)
    col5 = lax.broadcasted_iota(jnp.int32, (1, 5), 1)

    def _cond(st):
        return st[1]

    def _body(st):
        t, _ = st
        alive = alive_ref[...]
        ms = jnp.where(alive > f32(0.0), scores, f32(-1.0))
        mx = jnp.max(ms)
        j = jnp.max(jnp.where(ms == mx, idx, jnp.int32(-1)))
        pj = idx == j
        pz = f32(0.0)
        px1 = jnp.sum(jnp.where(pj, x1, pz))
        py1 = jnp.sum(jnp.where(pj, y1, pz))
        px2 = jnp.sum(jnp.where(pj, x2, pz))
        py2 = jnp.sum(jnp.where(pj, y2, pz))
        parea = jnp.sum(jnp.where(pj, area, pz))
        xx1 = jnp.maximum(x1, px1)
        yy1 = jnp.maximum(y1, py1)
        xx2 = jnp.minimum(x2, px2)
        yy2 = jnp.minimum(y2, py2)
        ww = jnp.maximum(xx2 - xx1, pz)
        hh = jnp.maximum(yy2 - yy1, pz)
        inter = ww * hh
        union = area - inter + parea
        iou = inter / union
        na = jnp.where((iou <= f32(NMS_THRESH)) & jnp.logical_not(pj), alive, pz)
        alive_ref[...] = na
        row = jnp.where(col5 == 0, mx,
              jnp.where(col5 == 1, px1,
              jnp.where(col5 == 2, py1,
              jnp.where(col5 == 3, px2, py2))))
        out_ref[0, 1, pl.ds(t, 1), :] = row
        t1 = t + 1
        go = (jnp.max(na) > f32(0.0)) & (t1 < TOP_K)
        return (t1, go)

    lax.while_loop(_cond, _body, (jnp.int32(0), go0))


@jax.jit
def kernel(loc_data, conf_data, prior_data):
    num = loc_data.shape[0]
    pad = _PAD_N - NUM_PRIORS
    loc_r = jnp.pad(loc_data.transpose(0, 2, 1), ((0, 0), (0, 0), (0, pad)))
    loc_r = loc_r.reshape(num, 4, _ROWS, 128)
    sc = conf_data.reshape(num, NUM_PRIORS, NUM_CLASSES)[:, :, 1]
    sc_r = jnp.pad(sc, ((0, 0), (0, pad))).reshape(num, 1, _ROWS, 128)
    pr_r = jnp.pad(prior_data.T, ((0, 0), (0, pad))).reshape(4, _ROWS, 128)

    out = pl.pallas_call(
        _nms_body,
        grid=(num,),
        in_specs=[
            pl.BlockSpec((1, 4, _ROWS, 128), lambda i: (i, 0, 0, 0)),
            pl.BlockSpec((1, 1, _ROWS, 128), lambda i: (i, 0, 0, 0)),
            pl.BlockSpec((4, _ROWS, 128), lambda i: (0, 0, 0)),
        ],
        out_specs=pl.BlockSpec((1, NUM_CLASSES, TOP_K, 5), lambda i: (i, 0, 0, 0)),
        out_shape=jax.ShapeDtypeStruct((num, NUM_CLASSES, TOP_K, 5), jnp.float32),
        scratch_shapes=[pltpu.VMEM((_ROWS, 128), jnp.float32)],
    )(loc_r, sc_r, pr_r)
    return out


# speedup vs baseline: 183.8572x; 1.0659x over previous
import functools

import jax
import jax.numpy as jnp
from jax import lax
from jax.experimental import pallas as pl
from jax.experimental.pallas import tpu as pltpu

NUM_CLASSES = 2
TOP_K = 750
NMS_THRESH = 0.3
CONF_THRESH = 0.05
NMS_TOP_K = 5000
NUM_PRIORS = 20000

_ROWS = 160
_PAD_N = _ROWS * 128


def _prep_batch(loc_ref, sc_ref, pr_ref, b, idx):
    f32 = jnp.float32
    l0 = loc_ref[b, 0]
    l1 = loc_ref[b, 1]
    l2 = loc_ref[b, 2]
    l3 = loc_ref[b, 3]
    pcx = pr_ref[0]
    pcy = pr_ref[1]
    pw = pr_ref[2]
    ph = pr_ref[3]
    cx = pcx + (l0 * f32(0.1)) * pw
    cy = pcy + (l1 * f32(0.1)) * ph
    w = pw * jnp.exp(l2 * f32(0.2))
    h = ph * jnp.exp(l3 * f32(0.2))
    x1 = cx + (-(w / f32(2.0)))
    y1 = cy + (-(h / f32(2.0)))
    x2 = w + x1
    y2 = h + y1
    area = (x2 - x1) * (y2 - y1)

    scores = sc_ref[b]
    valid = (scores > f32(CONF_THRESH)) & (idx < NUM_PRIORS)

    sbits = lax.bitcast_convert_type(scores, jnp.int32)
    sb = jnp.where(valid, sbits, jnp.int32(-1))

    def _bs_bits(_, lohi):
        lo, hi = lohi
        mid = (lo + hi) // 2
        cnt = jnp.sum(jnp.where(sb > mid, jnp.int32(1), jnp.int32(0)))
        pred = cnt < NMS_TOP_K
        return (jnp.where(pred, lo, mid + 1), jnp.where(pred, mid, hi))

    _, t_bits = lax.fori_loop(0, 31, _bs_bits,
                              (jnp.int32(0), jnp.int32(0x7F800000)))
    c_gt = jnp.sum(jnp.where(sb > t_bits, jnp.int32(1), jnp.int32(0)))
    r = NMS_TOP_K - c_gt
    eq = sb == t_bits

    def _bs_idx(_, lohi):
        lo, hi = lohi
        mid = (lo + hi) // 2
        cnt = jnp.sum(jnp.where(eq & (idx >= mid), jnp.int32(1), jnp.int32(0)))
        pred = cnt <= r
        return (jnp.where(pred, lo, mid + 1), jnp.where(pred, mid, hi))

    _, bound = lax.fori_loop(0, 16, _bs_idx, (jnp.int32(0), jnp.int32(_PAD_N)))
    participate = valid & ((sb > t_bits) | (eq & (idx >= bound)))
    alive0 = jnp.where(participate, f32(1.0), f32(0.0))
    return scores, x1, y1, x2, y2, area, alive0


def _nms_body(loc_ref, sc_ref, pr_ref, out_ref, a0_ref, a1_ref):
    f32 = jnp.float32
    idx = (lax.broadcasted_iota(jnp.int32, (_ROWS, 128), 0) * 128
           + lax.broadcasted_iota(jnp.int32, (_ROWS, 128), 1))

    data = []
    for b, aref in ((0, a0_ref), (1, a1_ref)):
        scores, x1, y1, x2, y2, area, alive0 = _prep_batch(
            loc_ref, sc_ref, pr_ref, b, idx)
        aref[...] = alive0
        data.append((scores, x1, y1, x2, y2, area, aref,
                     jnp.max(alive0) > f32(0.0)))

    out_ref[...] = jnp.zeros_like(out_ref)
    col5 = lax.broadcasted_iota(jnp.int32, (1, 5), 1)

    def _step(b, t, go_in):
        scores, x1, y1, x2, y2, area, aref, _ = data[b]
        alive = aref[...]
        ms = jnp.where(alive > f32(0.0), scores, f32(-1.0))
        mx = jnp.max(ms)
        j = jnp.max(jnp.where(ms == mx, idx, jnp.int32(-1)))
        pj = idx == j
        pz = f32(0.0)
        px1 = jnp.sum(jnp.where(pj, x1, pz))
        py1 = jnp.sum(jnp.where(pj, y1, pz))
        px2 = jnp.sum(jnp.where(pj, x2, pz))
        py2 = jnp.sum(jnp.where(pj, y2, pz))
        parea = jnp.sum(jnp.where(pj, area, pz))
        xx1 = jnp.maximum(x1, px1)
        yy1 = jnp.maximum(y1, py1)
        xx2 = jnp.minimum(x2, px2)
        yy2 = jnp.minimum(y2, py2)
        ww = jnp.maximum(xx2 - xx1, pz)
        hh = jnp.maximum(yy2 - yy1, pz)
        inter = ww * hh
        union = area - inter + parea
        iou = inter / union
        na = jnp.where((iou <= f32(NMS_THRESH)) & jnp.logical_not(pj),
                       alive, pz)
        aref[...] = jnp.where(go_in, na, alive)

        @pl.when(go_in)
        def _():
            row = jnp.where(col5 == 0, mx,
                  jnp.where(col5 == 1, px1,
                  jnp.where(col5 == 2, py1,
                  jnp.where(col5 == 3, px2, py2))))
            out_ref[b, 1, pl.ds(t, 1), :] = row

        t1 = jnp.where(go_in, t + 1, t)
        go_out = go_in & (jnp.max(na) > f32(0.0)) & (t1 < TOP_K)
        return t1, go_out

    def _cond(st):
        return st[2] | st[3]

    def _body(st):
        t0, t1, g0, g1 = st
        t0n, g0n = _step(0, t0, g0)
        t1n, g1n = _step(1, t1, g1)
        return (t0n, t1n, g0n, g1n)

    lax.while_loop(_cond, _body,
                   (jnp.int32(0), jnp.int32(0), data[0][7], data[1][7]))


@jax.jit
def kernel(loc_data, conf_data, prior_data):
    num = loc_data.shape[0]
    pad = _PAD_N - NUM_PRIORS
    loc_r = jnp.pad(loc_data.transpose(0, 2, 1), ((0, 0), (0, 0), (0, pad)))
    loc_r = loc_r.reshape(num, 4, _ROWS, 128)
    sc = conf_data.reshape(num, NUM_PRIORS, NUM_CLASSES)[:, :, 1]
    sc_r = jnp.pad(sc, ((0, 0), (0, pad))).reshape(num, _ROWS, 128)
    pr_r = jnp.pad(prior_data.T, ((0, 0), (0, pad))).reshape(4, _ROWS, 128)

    out = pl.pallas_call(
        _nms_body,
        out_shape=jax.ShapeDtypeStruct((num, NUM_CLASSES, TOP_K, 5),
                                       jnp.float32),
        scratch_shapes=[pltpu.VMEM((_ROWS, 128), jnp.float32),
                        pltpu.VMEM((_ROWS, 128), jnp.float32)],
    )(loc_r, sc_r, pr_r)
    return out
